# baseline (speedup 1.0000x reference)
import functools
import math

import jax
import jax.numpy as jnp
from jax import lax
from jax.experimental import pallas as pl
from jax.experimental.pallas import tpu as pltpu
from jax.experimental.pallas import tpu_sc as plsc

D = 128
D_E = 16
K = 10
H = 2
DQ = 2 * D
DK = 2 * D + D_E
NW = 32
CHUNK = 128
NB = 256


NBUF = 4


@functools.lru_cache(maxsize=None)
def _gather_call(specs, b_pad, tc_tiling):
    b_per_w = b_pad // NW
    n_chunks = b_per_w // CHUNK
    nt = len(specs)
    dtypes = [jnp.dtype(dn) for (_, _, dn) in specs]
    mesh = plsc.VectorSubcoreMesh(core_axis_name="c", subcore_axis_name="s")

    @functools.partial(
        pl.kernel,
        mesh=mesh,
        compiler_params=pltpu.CompilerParams(use_tc_tiling_on_sc=tc_tiling),
        out_type=[jax.ShapeDtypeStruct((b_pad, d), dt)
                  for (_, d, _), dt in zip(specs, dtypes)],
        scratch_types=(
            [pltpu.VMEM((n_chunks, CHUNK), jnp.int32)]
            + [pltpu.VMEM((2, CHUNK, d), dt) for (_, d, _), dt in zip(specs, dtypes)]
            + [pltpu.SemaphoreType.DMA]
        ),
    )
    def gk(*refs):
        tables = refs[:nt]
        idx_hbm = refs[nt]
        outs = refs[nt + 1:2 * nt + 1]
        idx_v = refs[2 * nt + 1]
        rows = refs[2 * nt + 2:3 * nt + 2]
        gsem = refs[3 * nt + 2]

        wid = lax.axis_index("s") * 2 + lax.axis_index("c")
        base = wid * b_per_w
        pltpu.sync_copy(idx_hbm.at[wid], idx_v)

        def fire(c, slot):
            for t in range(nt):
                pltpu.async_copy(tables[t].at[idx_v.at[c]], rows[t].at[slot], gsem)

        def wait_gather(c, slot):
            for t in range(nt):
                pltpu.make_async_copy(tables[t].at[idx_v.at[c]], rows[t].at[slot],
                                      gsem).wait()

        def store(c, slot):
            for t in range(nt):
                pltpu.sync_copy(rows[t].at[slot],
                                outs[t].at[pl.ds(base + c * CHUNK, CHUNK)])

        fire(0, 0)

        def body(c, carry):
            @pl.when(c + 1 < n_chunks)
            def _():
                fire(c + 1, lax.rem(c + 1, 2))

            wait_gather(c, lax.rem(c, 2))
            store(c, lax.rem(c, 2))
            return carry

        lax.fori_loop(0, n_chunks, body, 0)

    return gk


def _sc_gather(tables, idx, tc_tiling):
    b = idx.shape[0]
    b_pad = -(-b // (NW * CHUNK)) * (NW * CHUNK)
    idx = idx.astype(jnp.int32)
    if b_pad != b:
        idx = jnp.concatenate([idx, jnp.zeros((b_pad - b,), jnp.int32)])
    idx3 = idx.reshape(NW, b_pad // NW // CHUNK, CHUNK)
    specs = tuple((t.shape[0], t.shape[1], str(t.dtype)) for t in tables)
    fn = _gather_call(specs, b_pad, tc_tiling)
    res = fn(*tables, idx3)
    return res if isinstance(res, (list, tuple)) else [res]


_INV_2PI = 0.15915493667125702
_TWO_PI = 6.283185307179586


def _fast_cos(x):
    m = x * _INV_2PI
    y = m - jnp.round(m)
    q = jnp.round(2.0 * y)
    r = y - 0.5 * q
    z2 = (_TWO_PI * _TWO_PI) * (r * r)
    c = 1.0 + z2 * (-0.5 + z2 * (1.0 / 24.0 + z2 * (-1.0 / 720.0 + z2 * (1.0 / 40320.0))))
    return c * (1.0 - 2.0 * q * q)


@functools.lru_cache(maxsize=None)
def _transpose_pack_call(n_dims, n_edges):
    e_per_w = n_edges // NW
    cp = 2000
    n_pass = e_per_w // cp
    mesh = plsc.VectorSubcoreMesh(core_axis_name="c", subcore_axis_name="s")

    @functools.partial(
        pl.kernel,
        mesh=mesh,
        compiler_params=pltpu.CompilerParams(use_tc_tiling_on_sc=False,
                                             needs_layout_passes=False),
        out_type=jax.ShapeDtypeStruct((n_edges, n_dims), jnp.float32),
        scratch_types=[
            pltpu.VMEM((n_dims, cp), jnp.float32),
            pltpu.VMEM((cp, n_dims), jnp.float32),
        ],
    )
    def tk(et_hbm, out_hbm, buf, obuf):
        wid = lax.axis_index("s") * 2 + lax.axis_index("c")
        base = wid * e_per_w
        lane = lax.iota(jnp.int32, 16)

        def one_pass(pp, carry):
            e0 = base + pp * cp
            pltpu.sync_copy(et_hbm.at[:, pl.ds(e0, cp)], buf)

            def grp(g, carry2):
                eg = g * 16
                rows = eg + lane
                for d in range(n_dims):
                    v = buf[d, pl.ds(eg, 16)]
                    plsc.store_scatter(obuf, [rows, jnp.full((16,), d, jnp.int32)], v)
                return carry2

            lax.fori_loop(0, cp // 16, grp, 0)
            pltpu.sync_copy(obuf, out_hbm.at[pl.ds(e0, cp)])
            return carry

        lax.fori_loop(0, n_pass, one_pass, 0)

    return tk


def _transpose_pack(et):
    return _transpose_pack_call(et.shape[0], et.shape[1])(et)


def _concat_body(a_ref, b_ref, c_ref, out_ref):
    out_ref[:, 0:K] = a_ref[...]
    out_ref[:, K:2 * K] = b_ref[...]
    out_ref[:, 2 * K:3 * K] = lax.bitcast_convert_type(c_ref[...], jnp.int32)
    out_ref[:, 3 * K:] = jnp.zeros_like(out_ref[:, 3 * K:])


def _build_combo(nbr_nodes, nbr_edges, nbr_times):
    n_nodes = nbr_nodes.shape[0]
    rb = 4000
    return pl.pallas_call(
        _concat_body,
        grid=(n_nodes // rb,),
        in_specs=[pl.BlockSpec((rb, K), lambda i: (i, 0))] * 3,
        out_specs=pl.BlockSpec((rb, 32), lambda i: (i, 0)),
        out_shape=jax.ShapeDtypeStruct((n_nodes, 32), jnp.int32),
    )(nbr_nodes, nbr_edges, nbr_times)


def _attn_body(*refs, head):
    (src_ref, *rest) = refs
    nbr_refs = rest[:K]
    ef_refs = rest[K:2 * K]
    (ts_ref, nts_ref, wq_ref, wk_ref, wv_ref,
     wo_ref, f1w_ref, f1b_ref, f2w_ref, f2b_ref, tw_ref, tb_ref) = rest[2 * K:2 * K + 12]
    if head:
        hw_ref, hb_ref = rest[2 * K + 12:2 * K + 14]
        out_ref = rest[-1]
    else:
        out_ref = rest[-1]

    src = src_ref[...]
    wq = wq_ref[...]
    wk = wk_ref[...]
    wv = wv_ref[...]
    tw = tw_ref[...]
    tb = tb_ref[...]
    src_t = _fast_cos(tb)
    q = jnp.dot(src, wq[:D], preferred_element_type=jnp.float32) \
        + jnp.dot(src_t, wq[D:], preferred_element_type=jnp.float32)
    ts = ts_ref[...]
    nts = nts_ref[...]
    inv = 1.0 / math.sqrt(D)

    s0_cols, s1_cols, vks = [], [], []
    for k in range(K):
        e_t = _fast_cos((ts - nts[:, k:k + 1]) * tw + tb)
        nb_k = nbr_refs[k][...]
        ef_k = ef_refs[k][...]
        kk = (jnp.dot(nb_k, wk[:D], preferred_element_type=jnp.float32)
              + jnp.dot(e_t, wk[D:2 * D], preferred_element_type=jnp.float32)
              + jnp.dot(ef_k, wk[2 * D:], preferred_element_type=jnp.float32))
        vk = (jnp.dot(nb_k, wv[:D], preferred_element_type=jnp.float32)
              + jnp.dot(e_t, wv[D:2 * D], preferred_element_type=jnp.float32)
              + jnp.dot(ef_k, wv[2 * D:], preferred_element_type=jnp.float32))
        vks.append(vk)
        s0_cols.append(jnp.sum(kk[:, :D] * q[:, :D], axis=1, keepdims=True))
        s1_cols.append(jnp.sum(kk[:, D:] * q[:, D:], axis=1, keepdims=True))

    s0 = jnp.concatenate(s0_cols, axis=1) * inv
    s1 = jnp.concatenate(s1_cols, axis=1) * inv
    a0 = jax.nn.softmax(s0, axis=-1)
    a1 = jax.nn.softmax(s1, axis=-1)
    o0 = a0[:, 0:1] * vks[0][:, :D]
    o1 = a1[:, 0:1] * vks[0][:, D:]
    for k in range(1, K):
        o0 = o0 + a0[:, k:k + 1] * vks[k][:, :D]
        o1 = o1 + a1[:, k:k + 1] * vks[k][:, D:]

    wo = wo_ref[...]
    att = jnp.dot(o0, wo[:D], preferred_element_type=jnp.float32) \
        + jnp.dot(o1, wo[D:], preferred_element_type=jnp.float32)
    f1w = f1w_ref[...]
    h = jnp.dot(att, f1w[:DQ], preferred_element_type=jnp.float32) \
        + jnp.dot(src, f1w[DQ:], preferred_element_type=jnp.float32) + f1b_ref[...]
    h = jax.nn.relu(h)
    out = jnp.dot(h, f2w_ref[...], preferred_element_type=jnp.float32) + f2b_ref[...]
    if head:
        out_ref[...] = jax.nn.sigmoid(
            jnp.dot(out, hw_ref[...], preferred_element_type=jnp.float32) + hb_ref[...])
    else:
        out_ref[...] = out


def _attn_layer(feat, ef, ts_col, nts, weights, n, ef_base_blk,
                head_w=None, head_b=None):
    nb = n // NB
    wq, wk, wv, wo, f1w, f1b, f2w, f2b, tw, tb = weights
    head = head_w is not None

    in_specs = [pl.BlockSpec((NB, D), lambda i: (i, 0))]
    in_specs += [pl.BlockSpec((NB, D), (lambda i, k=k: ((k + 1) * nb + i, 0)))
                 for k in range(K)]
    in_specs += [pl.BlockSpec((NB, D_E), (lambda i, k=k: (ef_base_blk + k * nb + i, 0)))
                 for k in range(K)]
    in_specs += [
        pl.BlockSpec((NB, 1), lambda i: (i, 0)),
        pl.BlockSpec((NB, K), lambda i: (i, 0)),
        pl.BlockSpec(wq.shape, lambda i: (0, 0)),
        pl.BlockSpec(wk.shape, lambda i: (0, 0)),
        pl.BlockSpec(wv.shape, lambda i: (0, 0)),
        pl.BlockSpec(wo.shape, lambda i: (0, 0)),
        pl.BlockSpec(f1w.shape, lambda i: (0, 0)),
        pl.BlockSpec(f1b.shape, lambda i: (0, 0)),
        pl.BlockSpec(f2w.shape, lambda i: (0, 0)),
        pl.BlockSpec(f2b.shape, lambda i: (0, 0)),
        pl.BlockSpec(tw.shape, lambda i: (0, 0)),
        pl.BlockSpec(tb.shape, lambda i: (0, 0)),
    ]
    args = [feat] + [feat] * K + [ef] * K + [ts_col, nts, wq, wk, wv, wo,
                                             f1w, f1b, f2w, f2b, tw, tb]
    if head:
        in_specs += [pl.BlockSpec(head_w.shape, lambda i: (0, 0)),
                     pl.BlockSpec(head_b.shape, lambda i: (0, 0))]
        args += [head_w, head_b]
        d_out = 1
    else:
        d_out = D

    return pl.pallas_call(
        functools.partial(_attn_body, head=head),
        grid=(nb,),
        in_specs=in_specs,
        out_specs=pl.BlockSpec((NB, d_out), lambda i: (i, 0)),
        out_shape=jax.ShapeDtypeStruct((n, d_out), jnp.float32),
    )(*args)


def kernel(source_nodes, destination_nodes, edge_times, edge_idxs, n_neighbors,
           nbr_nodes, nbr_edges, nbr_times, params):
    p = params
    b2 = 2 * source_nodes.shape[0]
    nodes_all = jnp.concatenate([source_nodes, destination_nodes]).astype(jnp.int32)
    ts_all = jnp.concatenate([edge_times, edge_times])
    combo = _build_combo(nbr_nodes.astype(jnp.int32), nbr_edges.astype(jnp.int32),
                         nbr_times)

    g2, = _sc_gather((combo,), nodes_all, tc_tiling=False)
    g2 = g2[:b2]
    nbrs2 = g2[:, :K]
    neids2 = g2[:, K:2 * K]
    ntimes2 = lax.bitcast_convert_type(g2[:, 2 * K:3 * K], jnp.float32)

    n1 = b2 * (1 + K)
    l1_nodes = jnp.concatenate([nodes_all, nbrs2.T.reshape(-1)])
    l1_ts = jnp.concatenate([ts_all, ntimes2.T.reshape(-1)])

    g1, = _sc_gather((combo,), l1_nodes, tc_tiling=False)
    g1 = g1[:n1]
    nbrs1 = g1[:, :K]
    neids1 = g1[:, K:2 * K]
    ntimes1 = lax.bitcast_convert_type(g1[:, 2 * K:3 * K], jnp.float32)

    nf_idx = jnp.concatenate([l1_nodes, nbrs1.T.reshape(-1)])
    nf, = _sc_gather((p['node_features'],), nf_idx, tc_tiling=True)
    ef_idx = jnp.concatenate([neids1.T.reshape(-1), neids2.T.reshape(-1)])
    ef8t = _transpose_pack(p['edge_features'].T)
    ef, = _sc_gather((ef8t,), ef_idx, tc_tiling=False)

    def wts(l):
        return (p['Wq%d' % l], p['Wk%d' % l], p['Wv%d' % l], p['Wo%d' % l],
                p['fc1w%d' % l], p['fc1b%d' % l].reshape(1, D),
                p['fc2w%d' % l], p['fc2b%d' % l].reshape(1, D),
                p['time_w'].reshape(1, D), p['time_b'].reshape(1, D))

    emb1 = _attn_layer(nf, ef, l1_ts.reshape(-1, 1), ntimes1, wts(0),
                       n=n1, ef_base_blk=0)
    probs = _attn_layer(emb1, ef, ts_all.reshape(-1, 1), ntimes2, wts(1),
                        n=b2, ef_base_blk=(K * n1) // NB,
                        head_w=p['w_out'], head_b=p['b_out'].reshape(1, 1))
    nsrc = source_nodes.shape[0]
    return probs[:nsrc], probs[nsrc:]


# speedup vs baseline: 2.1943x; 2.1943x over previous
import functools
import math

import jax
import jax.numpy as jnp
from jax import lax
from jax.experimental import pallas as pl
from jax.experimental.pallas import tpu as pltpu
from jax.experimental.pallas import tpu_sc as plsc

D = 128
D_E = 16
K = 10
H = 2
DQ = 2 * D
DK = 2 * D + D_E
NW = 32
CHUNK = 128
NB = 256


NBUF = 4


@functools.lru_cache(maxsize=None)
def _gather_call(specs, b_pad, tc_tiling):
    b_per_w = b_pad // NW
    n_chunks = b_per_w // CHUNK
    nt = len(specs)
    dtypes = [jnp.dtype(dn) for (_, _, dn) in specs]
    mesh = plsc.VectorSubcoreMesh(core_axis_name="c", subcore_axis_name="s")

    @functools.partial(
        pl.kernel,
        mesh=mesh,
        compiler_params=pltpu.CompilerParams(use_tc_tiling_on_sc=tc_tiling),
        out_type=[jax.ShapeDtypeStruct((b_pad, d), dt)
                  for (_, d, _), dt in zip(specs, dtypes)],
        scratch_types=(
            [pltpu.VMEM((n_chunks, CHUNK), jnp.int32)]
            + [pltpu.VMEM((2, CHUNK, d), dt) for (_, d, _), dt in zip(specs, dtypes)]
            + [pltpu.SemaphoreType.DMA]
        ),
    )
    def gk(*refs):
        tables = refs[:nt]
        idx_hbm = refs[nt]
        outs = refs[nt + 1:2 * nt + 1]
        idx_v = refs[2 * nt + 1]
        rows = refs[2 * nt + 2:3 * nt + 2]
        gsem = refs[3 * nt + 2]

        wid = lax.axis_index("s") * 2 + lax.axis_index("c")
        base = wid * b_per_w
        pltpu.sync_copy(idx_hbm.at[wid], idx_v)

        def fire(c, slot):
            for t in range(nt):
                pltpu.async_copy(tables[t].at[idx_v.at[c]], rows[t].at[slot], gsem)

        def wait_gather(c, slot):
            for t in range(nt):
                pltpu.make_async_copy(tables[t].at[idx_v.at[c]], rows[t].at[slot],
                                      gsem).wait()

        def store(c, slot):
            for t in range(nt):
                pltpu.sync_copy(rows[t].at[slot],
                                outs[t].at[pl.ds(base + c * CHUNK, CHUNK)])

        fire(0, 0)

        def body(c, carry):
            @pl.when(c + 1 < n_chunks)
            def _():
                fire(c + 1, lax.rem(c + 1, 2))

            wait_gather(c, lax.rem(c, 2))
            store(c, lax.rem(c, 2))
            return carry

        lax.fori_loop(0, n_chunks, body, 0)

    return gk


def _sc_gather(tables, idx, tc_tiling):
    b = idx.shape[0]
    b_pad = -(-b // (NW * CHUNK)) * (NW * CHUNK)
    idx = idx.astype(jnp.int32)
    if b_pad != b:
        idx = jnp.concatenate([idx, jnp.zeros((b_pad - b,), jnp.int32)])
    idx3 = idx.reshape(NW, b_pad // NW // CHUNK, CHUNK)
    specs = tuple((t.shape[0], t.shape[1], str(t.dtype)) for t in tables)
    fn = _gather_call(specs, b_pad, tc_tiling)
    res = fn(*tables, idx3)
    return res if isinstance(res, (list, tuple)) else [res]


_INV_2PI = 0.15915493667125702
_TWO_PI = 6.283185307179586


def _fast_cos(x):
    m = x * _INV_2PI
    y = m - jnp.round(m)
    q = jnp.round(2.0 * y)
    r = y - 0.5 * q
    z2 = (_TWO_PI * _TWO_PI) * (r * r)
    c = 1.0 + z2 * (-0.5 + z2 * (1.0 / 24.0 + z2 * (-1.0 / 720.0 + z2 * (1.0 / 40320.0))))
    return c * (1.0 - 2.0 * q * q)


def _concat_body(a_ref, b_ref, c_ref, out_ref):
    out_ref[:, 0:K] = a_ref[...]
    out_ref[:, K:2 * K] = b_ref[...]
    out_ref[:, 2 * K:3 * K] = lax.bitcast_convert_type(c_ref[...], jnp.int32)
    out_ref[:, 3 * K:] = jnp.zeros_like(out_ref[:, 3 * K:])


def _build_combo(nbr_nodes, nbr_edges, nbr_times):
    n_nodes = nbr_nodes.shape[0]
    rb = 4000
    return pl.pallas_call(
        _concat_body,
        grid=(n_nodes // rb,),
        in_specs=[pl.BlockSpec((rb, K), lambda i: (i, 0))] * 3,
        out_specs=pl.BlockSpec((rb, 32), lambda i: (i, 0)),
        out_shape=jax.ShapeDtypeStruct((n_nodes, 32), jnp.int32),
    )(nbr_nodes, nbr_edges, nbr_times)


def _attn_body(*refs, head):
    (src_ref, *rest) = refs
    nbr_refs = rest[:K]
    ef_refs = rest[K:2 * K]
    (ts_ref, nts_ref, wq_ref, wk_ref, wv_ref,
     wo_ref, f1w_ref, f1b_ref, f2w_ref, f2b_ref, tw_ref, tb_ref) = rest[2 * K:2 * K + 12]
    if head:
        hw_ref, hb_ref = rest[2 * K + 12:2 * K + 14]
        out_ref = rest[-1]
    else:
        out_ref = rest[-1]

    src = src_ref[...]
    wq = wq_ref[...]
    wk = wk_ref[...]
    wv = wv_ref[...]
    tw = tw_ref[...]
    tb = tb_ref[...]
    src_t = _fast_cos(tb)
    q = jnp.dot(src, wq[:D], preferred_element_type=jnp.float32) \
        + jnp.dot(src_t, wq[D:], preferred_element_type=jnp.float32)
    ts = ts_ref[...]
    nts = nts_ref[...]
    inv = 1.0 / math.sqrt(D)

    s0_cols, s1_cols, vks = [], [], []
    for k in range(K):
        e_t = _fast_cos((ts - nts[:, k:k + 1]) * tw + tb)
        nb_k = nbr_refs[k][...]
        ef_k = ef_refs[k][...]
        kk = (jnp.dot(nb_k, wk[:D], preferred_element_type=jnp.float32)
              + jnp.dot(e_t, wk[D:2 * D], preferred_element_type=jnp.float32)
              + jnp.dot(ef_k, wk[2 * D:], preferred_element_type=jnp.float32))
        vk = (jnp.dot(nb_k, wv[:D], preferred_element_type=jnp.float32)
              + jnp.dot(e_t, wv[D:2 * D], preferred_element_type=jnp.float32)
              + jnp.dot(ef_k, wv[2 * D:], preferred_element_type=jnp.float32))
        vks.append(vk)
        s0_cols.append(jnp.sum(kk[:, :D] * q[:, :D], axis=1, keepdims=True))
        s1_cols.append(jnp.sum(kk[:, D:] * q[:, D:], axis=1, keepdims=True))

    s0 = jnp.concatenate(s0_cols, axis=1) * inv
    s1 = jnp.concatenate(s1_cols, axis=1) * inv
    a0 = jax.nn.softmax(s0, axis=-1)
    a1 = jax.nn.softmax(s1, axis=-1)
    o0 = a0[:, 0:1] * vks[0][:, :D]
    o1 = a1[:, 0:1] * vks[0][:, D:]
    for k in range(1, K):
        o0 = o0 + a0[:, k:k + 1] * vks[k][:, :D]
        o1 = o1 + a1[:, k:k + 1] * vks[k][:, D:]

    wo = wo_ref[...]
    att = jnp.dot(o0, wo[:D], preferred_element_type=jnp.float32) \
        + jnp.dot(o1, wo[D:], preferred_element_type=jnp.float32)
    f1w = f1w_ref[...]
    h = jnp.dot(att, f1w[:DQ], preferred_element_type=jnp.float32) \
        + jnp.dot(src, f1w[DQ:], preferred_element_type=jnp.float32) + f1b_ref[...]
    h = jax.nn.relu(h)
    out = jnp.dot(h, f2w_ref[...], preferred_element_type=jnp.float32) + f2b_ref[...]
    if head:
        out_ref[...] = jax.nn.sigmoid(
            jnp.dot(out, hw_ref[...], preferred_element_type=jnp.float32) + hb_ref[...])
    else:
        out_ref[...] = out


def _attn_layer(feat, ef, ts_col, nts, weights, n, ef_base_blk,
                head_w=None, head_b=None):
    nb = n // NB
    wq, wk, wv, wo, f1w, f1b, f2w, f2b, tw, tb = weights
    head = head_w is not None

    in_specs = [pl.BlockSpec((NB, D), lambda i: (i, 0))]
    in_specs += [pl.BlockSpec((NB, D), (lambda i, k=k: ((k + 1) * nb + i, 0)))
                 for k in range(K)]
    in_specs += [pl.BlockSpec((NB, D_E), (lambda i, k=k: (ef_base_blk + k * nb + i, 0)))
                 for k in range(K)]
    in_specs += [
        pl.BlockSpec((NB, 1), lambda i: (i, 0)),
        pl.BlockSpec((NB, K), lambda i: (i, 0)),
        pl.BlockSpec(wq.shape, lambda i: (0, 0)),
        pl.BlockSpec(wk.shape, lambda i: (0, 0)),
        pl.BlockSpec(wv.shape, lambda i: (0, 0)),
        pl.BlockSpec(wo.shape, lambda i: (0, 0)),
        pl.BlockSpec(f1w.shape, lambda i: (0, 0)),
        pl.BlockSpec(f1b.shape, lambda i: (0, 0)),
        pl.BlockSpec(f2w.shape, lambda i: (0, 0)),
        pl.BlockSpec(f2b.shape, lambda i: (0, 0)),
        pl.BlockSpec(tw.shape, lambda i: (0, 0)),
        pl.BlockSpec(tb.shape, lambda i: (0, 0)),
    ]
    args = [feat] + [feat] * K + [ef] * K + [ts_col, nts, wq, wk, wv, wo,
                                             f1w, f1b, f2w, f2b, tw, tb]
    if head:
        in_specs += [pl.BlockSpec(head_w.shape, lambda i: (0, 0)),
                     pl.BlockSpec(head_b.shape, lambda i: (0, 0))]
        args += [head_w, head_b]
        d_out = 1
    else:
        d_out = D

    return pl.pallas_call(
        functools.partial(_attn_body, head=head),
        grid=(nb,),
        in_specs=in_specs,
        out_specs=pl.BlockSpec((NB, d_out), lambda i: (i, 0)),
        out_shape=jax.ShapeDtypeStruct((n, d_out), jnp.float32),
    )(*args)


def kernel(source_nodes, destination_nodes, edge_times, edge_idxs, n_neighbors,
           nbr_nodes, nbr_edges, nbr_times, params):
    p = params
    b2 = 2 * source_nodes.shape[0]
    nodes_all = jnp.concatenate([source_nodes, destination_nodes]).astype(jnp.int32)
    ts_all = jnp.concatenate([edge_times, edge_times])
    combo = _build_combo(nbr_nodes.astype(jnp.int32), nbr_edges.astype(jnp.int32),
                         nbr_times)

    g2, = _sc_gather((combo,), nodes_all, tc_tiling=False)
    g2 = g2[:b2]
    nbrs2 = g2[:, :K]
    neids2 = g2[:, K:2 * K]
    ntimes2 = lax.bitcast_convert_type(g2[:, 2 * K:3 * K], jnp.float32)

    n1 = b2 * (1 + K)
    l1_nodes = jnp.concatenate([nodes_all, nbrs2.T.reshape(-1)])
    l1_ts = jnp.concatenate([ts_all, ntimes2.T.reshape(-1)])

    g1, = _sc_gather((combo,), l1_nodes, tc_tiling=False)
    g1 = g1[:n1]
    nbrs1 = g1[:, :K]
    neids1 = g1[:, K:2 * K]
    ntimes1 = lax.bitcast_convert_type(g1[:, 2 * K:3 * K], jnp.float32)

    nf_idx = jnp.concatenate([l1_nodes, nbrs1.T.reshape(-1)])
    nf, = _sc_gather((p['node_features'],), nf_idx, tc_tiling=True)
    ef_idx = jnp.concatenate([neids1.T.reshape(-1), neids2.T.reshape(-1)])
    ef, = _sc_gather((p['edge_features'],), ef_idx, tc_tiling=False)

    def wts(l):
        return (p['Wq%d' % l], p['Wk%d' % l], p['Wv%d' % l], p['Wo%d' % l],
                p['fc1w%d' % l], p['fc1b%d' % l].reshape(1, D),
                p['fc2w%d' % l], p['fc2b%d' % l].reshape(1, D),
                p['time_w'].reshape(1, D), p['time_b'].reshape(1, D))

    emb1 = _attn_layer(nf, ef, l1_ts.reshape(-1, 1), ntimes1, wts(0),
                       n=n1, ef_base_blk=0)
    probs = _attn_layer(emb1, ef, ts_all.reshape(-1, 1), ntimes2, wts(1),
                        n=b2, ef_base_blk=(K * n1) // NB,
                        head_w=p['w_out'], head_b=p['b_out'].reshape(1, 1))
    nsrc = source_nodes.shape[0]
    return probs[:nsrc], probs[nsrc:]


# speedup vs baseline: 2.2103x; 1.0073x over previous
import functools
import math

import jax
import jax.numpy as jnp
from jax import lax
from jax.experimental import pallas as pl
from jax.experimental.pallas import tpu as pltpu
from jax.experimental.pallas import tpu_sc as plsc

D = 128
D_E = 16
K = 10
H = 2
DQ = 2 * D
DK = 2 * D + D_E
NW = 32
CHUNK = 128
NB = 256


NBUF = 4


@functools.lru_cache(maxsize=None)
def _gather_call(specs, b_pad, tc_tiling):
    b_per_w = b_pad // NW
    n_chunks = b_per_w // CHUNK
    nt = len(specs)
    dtypes = [jnp.dtype(dn) for (_, _, dn) in specs]
    mesh = plsc.VectorSubcoreMesh(core_axis_name="c", subcore_axis_name="s")

    @functools.partial(
        pl.kernel,
        mesh=mesh,
        compiler_params=pltpu.CompilerParams(use_tc_tiling_on_sc=tc_tiling),
        out_type=[jax.ShapeDtypeStruct((b_pad, d), dt)
                  for (_, d, _), dt in zip(specs, dtypes)],
        scratch_types=(
            [pltpu.VMEM((n_chunks, CHUNK), jnp.int32)]
            + [pltpu.VMEM((NBUF, CHUNK, d), dt) for (_, d, _), dt in zip(specs, dtypes)]
            + [pltpu.SemaphoreType.DMA, pltpu.SemaphoreType.DMA]
        ),
    )
    def gk(*refs):
        tables = refs[:nt]
        idx_hbm = refs[nt]
        outs = refs[nt + 1:2 * nt + 1]
        idx_v = refs[2 * nt + 1]
        rows = refs[2 * nt + 2:3 * nt + 2]
        gsem, ssem = refs[3 * nt + 2:3 * nt + 4]

        wid = lax.axis_index("s") * 2 + lax.axis_index("c")
        base = wid * b_per_w
        pltpu.sync_copy(idx_hbm.at[wid], idx_v)

        def fire(c, slot):
            for t in range(nt):
                pltpu.async_copy(tables[t].at[idx_v.at[c]], rows[t].at[slot], gsem)

        def wait_gather(c, slot):
            for t in range(nt):
                pltpu.make_async_copy(tables[t].at[idx_v.at[c]], rows[t].at[slot],
                                      gsem).wait()

        def store(c, slot):
            for t in range(nt):
                pltpu.async_copy(rows[t].at[slot],
                                 outs[t].at[pl.ds(base + c * CHUNK, CHUNK)], ssem)

        def wait_store(c, slot):
            for t in range(nt):
                pltpu.make_async_copy(rows[t].at[slot],
                                      outs[t].at[pl.ds(base + c * CHUNK, CHUNK)],
                                      ssem).wait()

        for b in range(min(NBUF, n_chunks)):
            fire(b, b % NBUF)

        def body(c, carry):
            @pl.when(jnp.logical_and(c >= 1, c + NBUF - 1 < n_chunks))
            def _():
                wait_store(c - 1, (c - 1) % NBUF)
                fire(c + NBUF - 1, (c - 1) % NBUF)

            wait_gather(c, lax.rem(c, NBUF))
            store(c, lax.rem(c, NBUF))
            return carry

        lax.fori_loop(0, n_chunks, body, 0)
        for k in range(min(NBUF, n_chunks)):
            c = n_chunks - min(NBUF, n_chunks) + k
            wait_store(c, c % NBUF)

    return gk


def _sc_gather(tables, idx, tc_tiling):
    b = idx.shape[0]
    b_pad = -(-b // (NW * CHUNK)) * (NW * CHUNK)
    idx = idx.astype(jnp.int32)
    if b_pad != b:
        idx = jnp.concatenate([idx, jnp.zeros((b_pad - b,), jnp.int32)])
    idx3 = idx.reshape(NW, b_pad // NW // CHUNK, CHUNK)
    specs = tuple((t.shape[0], t.shape[1], str(t.dtype)) for t in tables)
    fn = _gather_call(specs, b_pad, tc_tiling)
    res = fn(*tables, idx3)
    return res if isinstance(res, (list, tuple)) else [res]


_INV_2PI = 0.15915493667125702
_TWO_PI = 6.283185307179586


def _fast_cos(x):
    m = x * _INV_2PI
    y = m - jnp.round(m)
    q = jnp.round(2.0 * y)
    r = y - 0.5 * q
    z2 = (_TWO_PI * _TWO_PI) * (r * r)
    c = 1.0 + z2 * (-0.5 + z2 * (1.0 / 24.0 + z2 * (-1.0 / 720.0 + z2 * (1.0 / 40320.0))))
    return c * (1.0 - 2.0 * q * q)


def _concat_body(a_ref, b_ref, c_ref, out_ref):
    out_ref[:, 0:K] = a_ref[...]
    out_ref[:, K:2 * K] = b_ref[...]
    out_ref[:, 2 * K:3 * K] = lax.bitcast_convert_type(c_ref[...], jnp.int32)
    out_ref[:, 3 * K:] = jnp.zeros_like(out_ref[:, 3 * K:])


def _build_combo(nbr_nodes, nbr_edges, nbr_times):
    n_nodes = nbr_nodes.shape[0]
    rb = 4000
    return pl.pallas_call(
        _concat_body,
        grid=(n_nodes // rb,),
        in_specs=[pl.BlockSpec((rb, K), lambda i: (i, 0))] * 3,
        out_specs=pl.BlockSpec((rb, 32), lambda i: (i, 0)),
        out_shape=jax.ShapeDtypeStruct((n_nodes, 32), jnp.int32),
    )(nbr_nodes, nbr_edges, nbr_times)


def _attn_body(*refs, head):
    (src_ref, *rest) = refs
    nbr_refs = rest[:K]
    ef_refs = rest[K:2 * K]
    (ts_ref, nts_ref, wq_ref, wk_ref, wv_ref,
     wo_ref, f1w_ref, f1b_ref, f2w_ref, f2b_ref, tw_ref, tb_ref) = rest[2 * K:2 * K + 12]
    if head:
        hw_ref, hb_ref = rest[2 * K + 12:2 * K + 14]
        out_ref = rest[-1]
    else:
        out_ref = rest[-1]

    src = src_ref[...]
    wq = wq_ref[...]
    wk = wk_ref[...]
    wv = wv_ref[...]
    tw = tw_ref[...]
    tb = tb_ref[...]
    src_t = _fast_cos(tb)
    q = jnp.dot(src, wq[:D], preferred_element_type=jnp.float32) \
        + jnp.dot(src_t, wq[D:], preferred_element_type=jnp.float32)
    ts = ts_ref[...]
    nts = nts_ref[...]
    inv = 1.0 / math.sqrt(D)

    s0_cols, s1_cols, vks = [], [], []
    for k in range(K):
        e_t = _fast_cos((ts - nts[:, k:k + 1]) * tw + tb)
        nb_k = nbr_refs[k][...]
        ef_k = ef_refs[k][...]
        kk = (jnp.dot(nb_k, wk[:D], preferred_element_type=jnp.float32)
              + jnp.dot(e_t, wk[D:2 * D], preferred_element_type=jnp.float32)
              + jnp.dot(ef_k, wk[2 * D:], preferred_element_type=jnp.float32))
        vk = (jnp.dot(nb_k, wv[:D], preferred_element_type=jnp.float32)
              + jnp.dot(e_t, wv[D:2 * D], preferred_element_type=jnp.float32)
              + jnp.dot(ef_k, wv[2 * D:], preferred_element_type=jnp.float32))
        vks.append(vk)
        s0_cols.append(jnp.sum(kk[:, :D] * q[:, :D], axis=1, keepdims=True))
        s1_cols.append(jnp.sum(kk[:, D:] * q[:, D:], axis=1, keepdims=True))

    s0 = jnp.concatenate(s0_cols, axis=1) * inv
    s1 = jnp.concatenate(s1_cols, axis=1) * inv
    a0 = jax.nn.softmax(s0, axis=-1)
    a1 = jax.nn.softmax(s1, axis=-1)
    o0 = a0[:, 0:1] * vks[0][:, :D]
    o1 = a1[:, 0:1] * vks[0][:, D:]
    for k in range(1, K):
        o0 = o0 + a0[:, k:k + 1] * vks[k][:, :D]
        o1 = o1 + a1[:, k:k + 1] * vks[k][:, D:]

    wo = wo_ref[...]
    att = jnp.dot(o0, wo[:D], preferred_element_type=jnp.float32) \
        + jnp.dot(o1, wo[D:], preferred_element_type=jnp.float32)
    f1w = f1w_ref[...]
    h = jnp.dot(att, f1w[:DQ], preferred_element_type=jnp.float32) \
        + jnp.dot(src, f1w[DQ:], preferred_element_type=jnp.float32) + f1b_ref[...]
    h = jax.nn.relu(h)
    out = jnp.dot(h, f2w_ref[...], preferred_element_type=jnp.float32) + f2b_ref[...]
    if head:
        out_ref[...] = jax.nn.sigmoid(
            jnp.dot(out, hw_ref[...], preferred_element_type=jnp.float32) + hb_ref[...])
    else:
        out_ref[...] = out


def _attn_layer(feat, ef, ts_col, nts, weights, n, ef_base_blk,
                head_w=None, head_b=None):
    nb = n // NB
    wq, wk, wv, wo, f1w, f1b, f2w, f2b, tw, tb = weights
    head = head_w is not None

    in_specs = [pl.BlockSpec((NB, D), lambda i: (i, 0))]
    in_specs += [pl.BlockSpec((NB, D), (lambda i, k=k: ((k + 1) * nb + i, 0)))
                 for k in range(K)]
    in_specs += [pl.BlockSpec((NB, D_E), (lambda i, k=k: (ef_base_blk + k * nb + i, 0)))
                 for k in range(K)]
    in_specs += [
        pl.BlockSpec((NB, 1), lambda i: (i, 0)),
        pl.BlockSpec((NB, K), lambda i: (i, 0)),
        pl.BlockSpec(wq.shape, lambda i: (0, 0)),
        pl.BlockSpec(wk.shape, lambda i: (0, 0)),
        pl.BlockSpec(wv.shape, lambda i: (0, 0)),
        pl.BlockSpec(wo.shape, lambda i: (0, 0)),
        pl.BlockSpec(f1w.shape, lambda i: (0, 0)),
        pl.BlockSpec(f1b.shape, lambda i: (0, 0)),
        pl.BlockSpec(f2w.shape, lambda i: (0, 0)),
        pl.BlockSpec(f2b.shape, lambda i: (0, 0)),
        pl.BlockSpec(tw.shape, lambda i: (0, 0)),
        pl.BlockSpec(tb.shape, lambda i: (0, 0)),
    ]
    args = [feat] + [feat] * K + [ef] * K + [ts_col, nts, wq, wk, wv, wo,
                                             f1w, f1b, f2w, f2b, tw, tb]
    if head:
        in_specs += [pl.BlockSpec(head_w.shape, lambda i: (0, 0)),
                     pl.BlockSpec(head_b.shape, lambda i: (0, 0))]
        args += [head_w, head_b]
        d_out = 1
    else:
        d_out = D

    return pl.pallas_call(
        functools.partial(_attn_body, head=head),
        grid=(nb,),
        in_specs=in_specs,
        out_specs=pl.BlockSpec((NB, d_out), lambda i: (i, 0)),
        out_shape=jax.ShapeDtypeStruct((n, d_out), jnp.float32),
    )(*args)


def kernel(source_nodes, destination_nodes, edge_times, edge_idxs, n_neighbors,
           nbr_nodes, nbr_edges, nbr_times, params):
    p = params
    b2 = 2 * source_nodes.shape[0]
    nodes_all = jnp.concatenate([source_nodes, destination_nodes]).astype(jnp.int32)
    ts_all = jnp.concatenate([edge_times, edge_times])
    combo = _build_combo(nbr_nodes.astype(jnp.int32), nbr_edges.astype(jnp.int32),
                         nbr_times)

    g2, = _sc_gather((combo,), nodes_all, tc_tiling=False)
    g2 = g2[:b2]
    nbrs2 = g2[:, :K]
    neids2 = g2[:, K:2 * K]
    ntimes2 = lax.bitcast_convert_type(g2[:, 2 * K:3 * K], jnp.float32)

    n1 = b2 * (1 + K)
    l1_nodes = jnp.concatenate([nodes_all, nbrs2.T.reshape(-1)])
    l1_ts = jnp.concatenate([ts_all, ntimes2.T.reshape(-1)])

    g1, = _sc_gather((combo,), l1_nodes, tc_tiling=False)
    g1 = g1[:n1]
    nbrs1 = g1[:, :K]
    neids1 = g1[:, K:2 * K]
    ntimes1 = lax.bitcast_convert_type(g1[:, 2 * K:3 * K], jnp.float32)

    nf_idx = jnp.concatenate([l1_nodes, nbrs1.T.reshape(-1)])
    nf, = _sc_gather((p['node_features'],), nf_idx, tc_tiling=True)
    ef_idx = jnp.concatenate([neids1.T.reshape(-1), neids2.T.reshape(-1)])
    ef, = _sc_gather((p['edge_features'],), ef_idx, tc_tiling=False)

    def wts(l):
        return (p['Wq%d' % l], p['Wk%d' % l], p['Wv%d' % l], p['Wo%d' % l],
                p['fc1w%d' % l], p['fc1b%d' % l].reshape(1, D),
                p['fc2w%d' % l], p['fc2b%d' % l].reshape(1, D),
                p['time_w'].reshape(1, D), p['time_b'].reshape(1, D))

    emb1 = _attn_layer(nf, ef, l1_ts.reshape(-1, 1), ntimes1, wts(0),
                       n=n1, ef_base_blk=0)
    probs = _attn_layer(emb1, ef, ts_all.reshape(-1, 1), ntimes2, wts(1),
                        n=b2, ef_base_blk=(K * n1) // NB,
                        head_w=p['w_out'], head_b=p['b_out'].reshape(1, 1))
    nsrc = source_nodes.shape[0]
    return probs[:nsrc], probs[nsrc:]
